# Initial kernel scaffold; baseline (speedup 1.0000x reference)
import jax
import jax.numpy as jnp
from jax.experimental import pallas as pl


def kernel(cond_logits, uncond_logits):
    raise NotImplementedError("write your pallas kernel here")


# speedup vs baseline: 1109.9185x; 1109.9185x over previous
import jax
import jax.numpy as jnp
import numpy as np
from jax import lax
from jax.experimental import pallas as pl
from jax.experimental.pallas import tpu as pltpu

CFG_SCALE = 2.0
TEMPERATURE = 0.85
TOP_P = 0.9
AUDIO_START_ID = 151669
AUDIO_END_ID = 215669
EOS_TOKEN_ID = 151645
SAMPLE_SEED = 1
B = 8
V = 1000000

W0 = 151552
WW = 65536
EOS_I = EOS_TOKEN_ID - W0
A0_I = AUDIO_START_ID - W0
A1_I = AUDIO_END_ID - W0
N_CHUNK = 16
BAND_C2 = 45056
OFF_C2 = 20480


def _rotl(x, d):
    return lax.shift_left(x, np.int32(d)) | lax.shift_right_logical(x, np.int32(32 - d))


def _threefry(x0, x1):
    ks0 = np.int32(0)
    ks1 = np.int32(SAMPLE_SEED)
    ks2 = np.int32(ks0 ^ ks1 ^ np.int32(0x1BD11BDA))
    rot1 = (13, 15, 26, 6)
    rot2 = (17, 29, 16, 24)
    x0 = x0 + ks0
    x1 = x1 + ks1
    for r in rot1:
        x0 = x0 + x1; x1 = _rotl(x1, r); x1 = x0 ^ x1
    x0 = x0 + ks1; x1 = x1 + ks2 + np.int32(1)
    for r in rot2:
        x0 = x0 + x1; x1 = _rotl(x1, r); x1 = x0 ^ x1
    x0 = x0 + ks2; x1 = x1 + ks0 + np.int32(2)
    for r in rot1:
        x0 = x0 + x1; x1 = _rotl(x1, r); x1 = x0 ^ x1
    x0 = x0 + ks0; x1 = x1 + ks1 + np.int32(3)
    for r in rot2:
        x0 = x0 + x1; x1 = _rotl(x1, r); x1 = x0 ^ x1
    x0 = x0 + ks1; x1 = x1 + ks2 + np.int32(4)
    for r in rot1:
        x0 = x0 + x1; x1 = _rotl(x1, r); x1 = x0 ^ x1
    x0 = x0 + ks2; x1 = x1 + ks0 + np.int32(5)
    return x0, x1


def _gumbel_window():
    row = lax.broadcasted_iota(jnp.int32, (B, WW), 0)
    col = lax.broadcasted_iota(jnp.int32, (B, WW), 1)
    flat = row * np.int32(V) + (col + np.int32(W0))
    o1, o2 = _threefry(jnp.zeros((B, WW), jnp.int32), flat)
    bits = o1 ^ o2
    fb = lax.shift_right_logical(bits, np.int32(9)) | np.int32(0x3F800000)
    f = lax.bitcast_convert_type(fb, jnp.float32) - jnp.float32(1.0)
    tiny = jnp.float32(np.finfo(np.float32).tiny)
    u = jnp.maximum(tiny, f * (jnp.float32(1.0) - tiny) + tiny)
    return -jnp.log(-jnp.log(u))


def _kernel(cond_ref, uncond_ref, probs_ref, ntok_ref, probs_w):
    c = pl.program_id(0)

    @pl.when(c == 0)
    def _compute():
        cw = cond_ref[...]
        uw = uncond_ref[...]
        cfg = uw + jnp.float32(CFG_SCALE) * (cw - uw)
        col = lax.broadcasted_iota(jnp.int32, (B, WW), 1)
        active = (col == EOS_I) | ((col >= A0_I) & (col < A1_I))
        neg = jnp.finfo(jnp.float32).min
        cfg = jnp.where(active, cfg, neg)
        m = jnp.max(cfg, axis=1, keepdims=True)
        p = jnp.where(active, jnp.exp(cfg - m), 0.0)
        Z = jnp.sum(p, axis=1, keepdims=True)
        target = jnp.float32(TOP_P) * Z

        pbits = lax.bitcast_convert_type(p, jnp.int32)
        lo0 = jnp.zeros((B, 1), jnp.int32)
        hi0 = jnp.full((B, 1), np.int32(0x3F800001), jnp.int32)

        def body(_, lohi):
            lo, hi = lohi
            mid = (lo >> 1) + (hi >> 1) + (lo & hi & 1)
            gm = jnp.sum(jnp.where(pbits >= mid, p, 0.0), axis=1, keepdims=True)
            gt = gm > target
            return jnp.where(gt, mid, lo), jnp.where(gt, hi, mid)

        lo, _ = lax.fori_loop(0, 31, body, (lo0, hi0))

        keep = active & (pbits >= lo)
        inv_t = jnp.float32(1.0 / TEMPERATURE)
        cfg2 = jnp.where(keep, cfg / jnp.float32(TEMPERATURE), -jnp.inf)
        p2 = jnp.where(keep, jnp.exp((cfg - m) * inv_t), 0.0)
        z2 = jnp.sum(p2, axis=1, keepdims=True)
        probs_w[...] = p2 / z2

        g = _gumbel_window()
        score = jnp.where(keep, cfg2 + g, -jnp.inf)
        smax = jnp.max(score, axis=1, keepdims=True)
        win = jnp.where(score == smax, col, np.int32(2 * WW))
        idx = jnp.min(win, axis=1, keepdims=True) + np.int32(W0)
        ntok_ref[...] = jnp.broadcast_to(idx, (B, 128))

    probs_ref[...] = jnp.zeros((B, WW), jnp.float32)

    @pl.when(c == 2)
    def _band_lo():
        probs_ref[:, OFF_C2:] = probs_w[:, :BAND_C2]

    @pl.when(c == 3)
    def _band_hi():
        probs_ref[:, :OFF_C2] = probs_w[:, BAND_C2:]


def kernel(cond_logits, uncond_logits):
    cond_w = lax.slice(cond_logits, (0, W0), (B, W0 + WW))
    uncond_w = lax.slice(uncond_logits, (0, W0), (B, W0 + WW))
    probs, ntok = pl.pallas_call(
        _kernel,
        grid=(N_CHUNK,),
        in_specs=[
            pl.BlockSpec((B, WW), lambda c: (0, 0)),
            pl.BlockSpec((B, WW), lambda c: (0, 0)),
        ],
        out_specs=[
            pl.BlockSpec((B, WW), lambda c: (0, c)),
            pl.BlockSpec((B, 128), lambda c: (0, 0)),
        ],
        out_shape=[
            jax.ShapeDtypeStruct((B, V), jnp.float32),
            jax.ShapeDtypeStruct((B, 128), jnp.int32),
        ],
        scratch_shapes=[
            pltpu.VMEM((B, WW), jnp.float32),
        ],
    )(cond_w, uncond_w)
    return probs, ntok[:, 0]
